# baseline (device time: 487879 ns/iter reference)
import jax
import jax.numpy as jnp
from jax import lax
from jax.experimental import pallas as pl
from jax.experimental.pallas import tpu as pltpu

P = 32
M = 2048
N = 2048
HN = N // 2
CH = M // P


def kernel(A, B):
    m, k = A.shape
    k2, n = B.shape

    def body(a_ref, b_ref, out_ref, rsp_buf, rsm_buf,
             rsp_send, rsp_recv, rsm_send, rsm_recv,
             agp_send, agp_recv, agm_send, agm_recv):
        my = lax.axis_index("i")
        left = lax.rem(my + P - 1, P)
        right = lax.rem(my + 1, P)

        def rows(c):
            return pl.ds(c * CH, CH)

        out_ref[rows(my), :] = jnp.dot(
            a_ref[rows(my), :], b_ref[...], preferred_element_type=jnp.float32
        )

        barrier = pltpu.get_barrier_semaphore()
        for nbr in (left, right):
            pl.semaphore_signal(
                barrier, inc=1, device_id=(nbr,),
                device_id_type=pl.DeviceIdType.MESH,
            )
        pl.semaphore_wait(barrier, 2)

        for s in range(P - 1):
            scp = lax.rem(my - s + P, P)
            rcp = lax.rem(my - s - 1 + P, P)
            scm = lax.rem(my + s, P)
            rcm = lax.rem(my + s + 1, P)
            sp = pltpu.make_async_remote_copy(
                src_ref=out_ref.at[rows(scp), pl.ds(0, HN)],
                dst_ref=rsp_buf.at[s],
                send_sem=rsp_send.at[s],
                recv_sem=rsp_recv.at[s],
                device_id=(right,),
                device_id_type=pl.DeviceIdType.MESH,
            )
            sm = pltpu.make_async_remote_copy(
                src_ref=out_ref.at[rows(scm), pl.ds(HN, HN)],
                dst_ref=rsm_buf.at[s],
                send_sem=rsm_send.at[s],
                recv_sem=rsm_recv.at[s],
                device_id=(left,),
                device_id_type=pl.DeviceIdType.MESH,
            )
            sp.start()
            sm.start()
            tile_p = jnp.dot(
                a_ref[rows(rcp), :], b_ref[:, 0:HN],
                preferred_element_type=jnp.float32,
            )
            tile_m = jnp.dot(
                a_ref[rows(rcm), :], b_ref[:, HN:N],
                preferred_element_type=jnp.float32,
            )
            rp = pltpu.make_async_remote_copy(
                src_ref=rsp_buf.at[s],
                dst_ref=rsp_buf.at[s],
                send_sem=rsp_send.at[s],
                recv_sem=rsp_recv.at[s],
                device_id=(right,),
                device_id_type=pl.DeviceIdType.MESH,
            )
            rm = pltpu.make_async_remote_copy(
                src_ref=rsm_buf.at[s],
                dst_ref=rsm_buf.at[s],
                send_sem=rsm_send.at[s],
                recv_sem=rsm_recv.at[s],
                device_id=(left,),
                device_id_type=pl.DeviceIdType.MESH,
            )
            rp.wait_recv()
            out_ref[rows(rcp), pl.ds(0, HN)] = tile_p + rsp_buf[s]
            rm.wait_recv()
            out_ref[rows(rcm), pl.ds(HN, HN)] = tile_m + rsm_buf[s]

        for s in range(P - 1):
            for buf, sems, rsems, nbr in (
                (rsp_buf, rsp_send, rsp_recv, right),
                (rsm_buf, rsm_send, rsm_recv, left),
            ):
                d = pltpu.make_async_remote_copy(
                    src_ref=buf.at[s],
                    dst_ref=buf.at[s],
                    send_sem=sems.at[s],
                    recv_sem=rsems.at[s],
                    device_id=(nbr,),
                    device_id_type=pl.DeviceIdType.MESH,
                )
                d.wait_send()

        for s in range(P - 1):
            scp = lax.rem(my + 1 - s + P, P)
            rcp = lax.rem(my - s + P, P)
            scm = lax.rem(my - 1 + s + P, P)
            rcm = lax.rem(my + s, P)
            sp = pltpu.make_async_remote_copy(
                src_ref=out_ref.at[rows(scp), pl.ds(0, HN)],
                dst_ref=out_ref.at[rows(scp), pl.ds(0, HN)],
                send_sem=agp_send.at[s],
                recv_sem=agp_recv.at[s],
                device_id=(right,),
                device_id_type=pl.DeviceIdType.MESH,
            )
            sm = pltpu.make_async_remote_copy(
                src_ref=out_ref.at[rows(scm), pl.ds(HN, HN)],
                dst_ref=out_ref.at[rows(scm), pl.ds(HN, HN)],
                send_sem=agm_send.at[s],
                recv_sem=agm_recv.at[s],
                device_id=(left,),
                device_id_type=pl.DeviceIdType.MESH,
            )
            sp.start()
            sm.start()
            rp = pltpu.make_async_remote_copy(
                src_ref=out_ref.at[rows(rcp), pl.ds(0, HN)],
                dst_ref=out_ref.at[rows(rcp), pl.ds(0, HN)],
                send_sem=agp_send.at[s],
                recv_sem=agp_recv.at[s],
                device_id=(right,),
                device_id_type=pl.DeviceIdType.MESH,
            )
            rm = pltpu.make_async_remote_copy(
                src_ref=out_ref.at[rows(rcm), pl.ds(HN, HN)],
                dst_ref=out_ref.at[rows(rcm), pl.ds(HN, HN)],
                send_sem=agm_send.at[s],
                recv_sem=agm_recv.at[s],
                device_id=(left,),
                device_id_type=pl.DeviceIdType.MESH,
            )
            rp.wait_recv()
            rm.wait_recv()

        for s in range(P - 1):
            for buf, sems, rsems, nbr in (
                (rsp_buf, agp_send, agp_recv, right),
                (rsm_buf, agm_send, agm_recv, left),
            ):
                d = pltpu.make_async_remote_copy(
                    src_ref=buf.at[s],
                    dst_ref=buf.at[s],
                    send_sem=sems.at[s],
                    recv_sem=rsems.at[s],
                    device_id=(nbr,),
                    device_id_type=pl.DeviceIdType.MESH,
                )
                d.wait_send()

    nsem = P - 1
    return pl.pallas_call(
        body,
        out_shape=jax.ShapeDtypeStruct((m, n), jnp.float32),
        in_specs=[
            pl.BlockSpec(memory_space=pltpu.VMEM),
            pl.BlockSpec(memory_space=pltpu.VMEM),
        ],
        out_specs=pl.BlockSpec(memory_space=pltpu.VMEM),
        scratch_shapes=[
            pltpu.VMEM((nsem, CH, HN), jnp.float32),
            pltpu.VMEM((nsem, CH, HN), jnp.float32),
            pltpu.SemaphoreType.DMA((nsem,)),
            pltpu.SemaphoreType.DMA((nsem,)),
            pltpu.SemaphoreType.DMA((nsem,)),
            pltpu.SemaphoreType.DMA((nsem,)),
            pltpu.SemaphoreType.DMA((nsem,)),
            pltpu.SemaphoreType.DMA((nsem,)),
            pltpu.SemaphoreType.DMA((nsem,)),
            pltpu.SemaphoreType.DMA((nsem,)),
        ],
        compiler_params=pltpu.CompilerParams(
            collective_id=0,
            vmem_limit_bytes=100 * 1024 * 1024,
        ),
    )(A, B)


# device time: 381295 ns/iter; 1.2795x vs baseline; 1.2795x over previous
import jax
import jax.numpy as jnp
from jax import lax
from jax.experimental import pallas as pl
from jax.experimental.pallas import tpu as pltpu

P = 32
M = 2048
N = 2048
HN = N // 2
CH = M // P
F16 = jnp.bfloat16


def kernel(A, B):
    m, k = A.shape
    k2, n = B.shape

    def body(a_ref, b_ref, out_ref,
             ssp, ssm, rsp_buf, rsm_buf, agp_buf, agm_buf,
             rsp_send, rsp_recv, rsm_send, rsm_recv,
             agp_send, agp_recv, agm_send, agm_recv):
        my = lax.axis_index("i")
        left = lax.rem(my + P - 1, P)
        right = lax.rem(my + 1, P)

        def rows(c):
            return pl.ds(c * CH, CH)

        out_ref[...] = jnp.dot(
            a_ref[...], b_ref[...], preferred_element_type=jnp.float32
        )
        ssp[0] = out_ref[rows(my), pl.ds(0, HN)].astype(F16)
        ssm[0] = out_ref[rows(my), pl.ds(HN, HN)].astype(F16)

        barrier = pltpu.get_barrier_semaphore()
        for nbr in (left, right):
            pl.semaphore_signal(
                barrier, inc=1, device_id=(nbr,),
                device_id_type=pl.DeviceIdType.MESH,
            )
        pl.semaphore_wait(barrier, 2)

        for s in range(P - 1):
            rcp = lax.rem(my - s - 1 + P, P)
            rcm = lax.rem(my + s + 1, P)
            sp = pltpu.make_async_remote_copy(
                src_ref=ssp.at[s],
                dst_ref=rsp_buf.at[s],
                send_sem=rsp_send.at[s],
                recv_sem=rsp_recv.at[s],
                device_id=(right,),
                device_id_type=pl.DeviceIdType.MESH,
            )
            sm = pltpu.make_async_remote_copy(
                src_ref=ssm.at[s],
                dst_ref=rsm_buf.at[s],
                send_sem=rsm_send.at[s],
                recv_sem=rsm_recv.at[s],
                device_id=(left,),
                device_id_type=pl.DeviceIdType.MESH,
            )
            sp.start()
            sm.start()
            sp.wait_recv()
            acc_p = out_ref[rows(rcp), pl.ds(0, HN)] + rsp_buf[s].astype(
                jnp.float32
            )
            out_ref[rows(rcp), pl.ds(0, HN)] = acc_p
            ssp[s + 1] = acc_p.astype(F16)
            sm.wait_recv()
            acc_m = out_ref[rows(rcm), pl.ds(HN, HN)] + rsm_buf[s].astype(
                jnp.float32
            )
            out_ref[rows(rcm), pl.ds(HN, HN)] = acc_m
            ssm[s + 1] = acc_m.astype(F16)

        for s in range(P - 1):
            for stg, sems, rsems, buf, nbr in (
                (ssp, rsp_send, rsp_recv, rsp_buf, right),
                (ssm, rsm_send, rsm_recv, rsm_buf, left),
            ):
                d = pltpu.make_async_remote_copy(
                    src_ref=stg.at[s],
                    dst_ref=buf.at[s],
                    send_sem=sems.at[s],
                    recv_sem=rsems.at[s],
                    device_id=(nbr,),
                    device_id_type=pl.DeviceIdType.MESH,
                )
                d.wait_send()

        for s in range(P - 1):
            rcp = lax.rem(my - s + P, P)
            rcm = lax.rem(my + s, P)
            srcp = ssp.at[P - 1] if s == 0 else agp_buf.at[s - 1]
            srcm = ssm.at[P - 1] if s == 0 else agm_buf.at[s - 1]
            sp = pltpu.make_async_remote_copy(
                src_ref=srcp,
                dst_ref=agp_buf.at[s],
                send_sem=agp_send.at[s],
                recv_sem=agp_recv.at[s],
                device_id=(right,),
                device_id_type=pl.DeviceIdType.MESH,
            )
            sm = pltpu.make_async_remote_copy(
                src_ref=srcm,
                dst_ref=agm_buf.at[s],
                send_sem=agm_send.at[s],
                recv_sem=agm_recv.at[s],
                device_id=(left,),
                device_id_type=pl.DeviceIdType.MESH,
            )
            sp.start()
            sm.start()
            sp.wait_recv()
            out_ref[rows(rcp), pl.ds(0, HN)] = agp_buf[s].astype(jnp.float32)
            sm.wait_recv()
            out_ref[rows(rcm), pl.ds(HN, HN)] = agm_buf[s].astype(jnp.float32)

        for s in range(P - 1):
            for sems, rsems, buf, nbr in (
                (agp_send, agp_recv, agp_buf, right),
                (agm_send, agm_recv, agm_buf, left),
            ):
                d = pltpu.make_async_remote_copy(
                    src_ref=buf.at[s],
                    dst_ref=buf.at[s],
                    send_sem=sems.at[s],
                    recv_sem=rsems.at[s],
                    device_id=(nbr,),
                    device_id_type=pl.DeviceIdType.MESH,
                )
                d.wait_send()

    nsem = P - 1
    return pl.pallas_call(
        body,
        out_shape=jax.ShapeDtypeStruct((m, n), jnp.float32),
        in_specs=[
            pl.BlockSpec(memory_space=pltpu.VMEM),
            pl.BlockSpec(memory_space=pltpu.VMEM),
        ],
        out_specs=pl.BlockSpec(memory_space=pltpu.VMEM),
        scratch_shapes=[
            pltpu.VMEM((P, CH, HN), F16),
            pltpu.VMEM((P, CH, HN), F16),
            pltpu.VMEM((nsem, CH, HN), F16),
            pltpu.VMEM((nsem, CH, HN), F16),
            pltpu.VMEM((nsem, CH, HN), F16),
            pltpu.VMEM((nsem, CH, HN), F16),
            pltpu.SemaphoreType.DMA((nsem,)),
            pltpu.SemaphoreType.DMA((nsem,)),
            pltpu.SemaphoreType.DMA((nsem,)),
            pltpu.SemaphoreType.DMA((nsem,)),
            pltpu.SemaphoreType.DMA((nsem,)),
            pltpu.SemaphoreType.DMA((nsem,)),
            pltpu.SemaphoreType.DMA((nsem,)),
            pltpu.SemaphoreType.DMA((nsem,)),
        ],
        compiler_params=pltpu.CompilerParams(
            collective_id=0,
            vmem_limit_bytes=100 * 1024 * 1024,
        ),
    )(A, B)


# device time: 380928 ns/iter; 1.2808x vs baseline; 1.0010x over previous
import jax
import jax.numpy as jnp
from jax import lax
from jax.experimental import pallas as pl
from jax.experimental.pallas import tpu as pltpu

P = 32
M = 2048
N = 2048
HN = N // 2
CH = M // P
F16 = jnp.bfloat16


def kernel(A, B):
    m, k = A.shape
    k2, n = B.shape

    def body(a_ref, b_ref, out_ref,
             a_bf, b_bf,
             ssp, ssm, rsp_buf, rsm_buf, agp_buf, agm_buf,
             rsp_send, rsp_recv, rsm_send, rsm_recv,
             agp_send, agp_recv, agm_send, agm_recv):
        my = lax.axis_index("i")
        left = lax.rem(my + P - 1, P)
        right = lax.rem(my + 1, P)

        def rows(c):
            return pl.ds(c * CH, CH)

        a_bf[...] = a_ref[...].astype(F16)
        b_bf[...] = b_ref[...].astype(F16)
        out_ref[...] = jnp.dot(
            a_bf[...], b_bf[...], preferred_element_type=jnp.float32
        )
        ssp[0] = out_ref[rows(my), pl.ds(0, HN)].astype(F16)
        ssm[0] = out_ref[rows(my), pl.ds(HN, HN)].astype(F16)

        barrier = pltpu.get_barrier_semaphore()
        for nbr in (left, right):
            pl.semaphore_signal(
                barrier, inc=1, device_id=(nbr,),
                device_id_type=pl.DeviceIdType.MESH,
            )
        pl.semaphore_wait(barrier, 2)

        for s in range(P - 1):
            rcp = lax.rem(my - s - 1 + P, P)
            rcm = lax.rem(my + s + 1, P)
            sp = pltpu.make_async_remote_copy(
                src_ref=ssp.at[s],
                dst_ref=rsp_buf.at[s],
                send_sem=rsp_send.at[s],
                recv_sem=rsp_recv.at[s],
                device_id=(right,),
                device_id_type=pl.DeviceIdType.MESH,
            )
            sm = pltpu.make_async_remote_copy(
                src_ref=ssm.at[s],
                dst_ref=rsm_buf.at[s],
                send_sem=rsm_send.at[s],
                recv_sem=rsm_recv.at[s],
                device_id=(left,),
                device_id_type=pl.DeviceIdType.MESH,
            )
            sp.start()
            sm.start()
            sp.wait_recv()
            acc_p = out_ref[rows(rcp), pl.ds(0, HN)] + rsp_buf[s].astype(
                jnp.float32
            )
            out_ref[rows(rcp), pl.ds(0, HN)] = acc_p
            ssp[s + 1] = acc_p.astype(F16)
            sm.wait_recv()
            acc_m = out_ref[rows(rcm), pl.ds(HN, HN)] + rsm_buf[s].astype(
                jnp.float32
            )
            out_ref[rows(rcm), pl.ds(HN, HN)] = acc_m
            ssm[s + 1] = acc_m.astype(F16)

        for s in range(P - 1):
            for stg, sems, rsems, buf, nbr in (
                (ssp, rsp_send, rsp_recv, rsp_buf, right),
                (ssm, rsm_send, rsm_recv, rsm_buf, left),
            ):
                d = pltpu.make_async_remote_copy(
                    src_ref=stg.at[s],
                    dst_ref=buf.at[s],
                    send_sem=sems.at[s],
                    recv_sem=rsems.at[s],
                    device_id=(nbr,),
                    device_id_type=pl.DeviceIdType.MESH,
                )
                d.wait_send()

        for s in range(P - 1):
            rcp = lax.rem(my - s + P, P)
            rcm = lax.rem(my + s, P)
            srcp = ssp.at[P - 1] if s == 0 else agp_buf.at[s - 1]
            srcm = ssm.at[P - 1] if s == 0 else agm_buf.at[s - 1]
            sp = pltpu.make_async_remote_copy(
                src_ref=srcp,
                dst_ref=agp_buf.at[s],
                send_sem=agp_send.at[s],
                recv_sem=agp_recv.at[s],
                device_id=(right,),
                device_id_type=pl.DeviceIdType.MESH,
            )
            sm = pltpu.make_async_remote_copy(
                src_ref=srcm,
                dst_ref=agm_buf.at[s],
                send_sem=agm_send.at[s],
                recv_sem=agm_recv.at[s],
                device_id=(left,),
                device_id_type=pl.DeviceIdType.MESH,
            )
            sp.start()
            sm.start()
            sp.wait_recv()
            out_ref[rows(rcp), pl.ds(0, HN)] = agp_buf[s].astype(jnp.float32)
            sm.wait_recv()
            out_ref[rows(rcm), pl.ds(HN, HN)] = agm_buf[s].astype(jnp.float32)

        for s in range(P - 1):
            for sems, rsems, buf, nbr in (
                (agp_send, agp_recv, agp_buf, right),
                (agm_send, agm_recv, agm_buf, left),
            ):
                d = pltpu.make_async_remote_copy(
                    src_ref=buf.at[s],
                    dst_ref=buf.at[s],
                    send_sem=sems.at[s],
                    recv_sem=rsems.at[s],
                    device_id=(nbr,),
                    device_id_type=pl.DeviceIdType.MESH,
                )
                d.wait_send()

    nsem = P - 1
    return pl.pallas_call(
        body,
        out_shape=jax.ShapeDtypeStruct((m, n), jnp.float32),
        in_specs=[
            pl.BlockSpec(memory_space=pltpu.VMEM),
            pl.BlockSpec(memory_space=pltpu.VMEM),
        ],
        out_specs=pl.BlockSpec(memory_space=pltpu.VMEM),
        scratch_shapes=[
            pltpu.VMEM((M, k), F16),
            pltpu.VMEM((k, N), F16),
            pltpu.VMEM((P, CH, HN), F16),
            pltpu.VMEM((P, CH, HN), F16),
            pltpu.VMEM((nsem, CH, HN), F16),
            pltpu.VMEM((nsem, CH, HN), F16),
            pltpu.VMEM((nsem, CH, HN), F16),
            pltpu.VMEM((nsem, CH, HN), F16),
            pltpu.SemaphoreType.DMA((nsem,)),
            pltpu.SemaphoreType.DMA((nsem,)),
            pltpu.SemaphoreType.DMA((nsem,)),
            pltpu.SemaphoreType.DMA((nsem,)),
            pltpu.SemaphoreType.DMA((nsem,)),
            pltpu.SemaphoreType.DMA((nsem,)),
            pltpu.SemaphoreType.DMA((nsem,)),
            pltpu.SemaphoreType.DMA((nsem,)),
        ],
        compiler_params=pltpu.CompilerParams(
            collective_id=0,
            vmem_limit_bytes=100 * 1024 * 1024,
        ),
    )(A, B)


# device time: 380666 ns/iter; 1.2816x vs baseline; 1.0007x over previous
import jax
import jax.numpy as jnp
from jax import lax
from jax.experimental import pallas as pl
from jax.experimental.pallas import tpu as pltpu

P = 32
M = 2048
N = 2048
HN = N // 2
CH = M // P
F16 = jnp.bfloat16


def kernel(A, B):
    m, k = A.shape
    k2, n = B.shape

    def body(a_ref, b_ref, out_ref,
             a_bf, b_bf,
             ssp, ssm, rsp_buf, rsm_buf, agp_buf, agm_buf,
             rsp_send, rsp_recv, rsm_send, rsm_recv,
             agp_send, agp_recv, agm_send, agm_recv):
        my = lax.axis_index("i")
        left = lax.rem(my + P - 1, P)
        right = lax.rem(my + 1, P)

        def rows(c):
            return pl.ds(c * CH, CH)

        a_bf[...] = a_ref[...].astype(F16)
        b_bf[...] = b_ref[...].astype(F16)
        out_ref[...] = jnp.dot(
            a_bf[...], b_bf[...], preferred_element_type=jnp.float32
        )
        ssp[0] = out_ref[rows(my), pl.ds(0, HN)].astype(F16)
        ssm[0] = out_ref[rows(my), pl.ds(HN, HN)].astype(F16)

        barrier = pltpu.get_barrier_semaphore()
        for nbr in (left, right):
            pl.semaphore_signal(
                barrier, inc=1, device_id=(nbr,),
                device_id_type=pl.DeviceIdType.MESH,
            )
        pl.semaphore_wait(barrier, 2)

        for s in range(P - 1):
            rcp = lax.rem(my - s - 1 + P, P)
            rcm = lax.rem(my + s + 1, P)
            sp = pltpu.make_async_remote_copy(
                src_ref=ssp.at[s],
                dst_ref=rsp_buf.at[s],
                send_sem=rsp_send.at[s],
                recv_sem=rsp_recv.at[s],
                device_id=(right,),
                device_id_type=pl.DeviceIdType.MESH,
            )
            sm = pltpu.make_async_remote_copy(
                src_ref=ssm.at[s],
                dst_ref=rsm_buf.at[s],
                send_sem=rsm_send.at[s],
                recv_sem=rsm_recv.at[s],
                device_id=(left,),
                device_id_type=pl.DeviceIdType.MESH,
            )
            sp.start()
            sm.start()
            sp.wait_recv()
            acc_p = out_ref[rows(rcp), pl.ds(0, HN)] + rsp_buf[s].astype(
                jnp.float32
            )
            ssp[s + 1] = acc_p.astype(F16)
            if s == P - 2:
                out_ref[rows(rcp), pl.ds(0, HN)] = acc_p
            sm.wait_recv()
            acc_m = out_ref[rows(rcm), pl.ds(HN, HN)] + rsm_buf[s].astype(
                jnp.float32
            )
            ssm[s + 1] = acc_m.astype(F16)
            if s == P - 2:
                out_ref[rows(rcm), pl.ds(HN, HN)] = acc_m

        for s in range(P - 1):
            for stg, sems, rsems, buf, nbr in (
                (ssp, rsp_send, rsp_recv, rsp_buf, right),
                (ssm, rsm_send, rsm_recv, rsm_buf, left),
            ):
                d = pltpu.make_async_remote_copy(
                    src_ref=stg.at[s],
                    dst_ref=buf.at[s],
                    send_sem=sems.at[s],
                    recv_sem=rsems.at[s],
                    device_id=(nbr,),
                    device_id_type=pl.DeviceIdType.MESH,
                )
                d.wait_send()

        for s in range(P - 1):
            rcp = lax.rem(my - s + P, P)
            rcm = lax.rem(my + s, P)
            srcp = ssp.at[P - 1] if s == 0 else agp_buf.at[s - 1]
            srcm = ssm.at[P - 1] if s == 0 else agm_buf.at[s - 1]
            sp = pltpu.make_async_remote_copy(
                src_ref=srcp,
                dst_ref=agp_buf.at[s],
                send_sem=agp_send.at[s],
                recv_sem=agp_recv.at[s],
                device_id=(right,),
                device_id_type=pl.DeviceIdType.MESH,
            )
            sm = pltpu.make_async_remote_copy(
                src_ref=srcm,
                dst_ref=agm_buf.at[s],
                send_sem=agm_send.at[s],
                recv_sem=agm_recv.at[s],
                device_id=(left,),
                device_id_type=pl.DeviceIdType.MESH,
            )
            sp.start()
            sm.start()
            sp.wait_recv()
            out_ref[rows(rcp), pl.ds(0, HN)] = agp_buf[s].astype(jnp.float32)
            sm.wait_recv()
            out_ref[rows(rcm), pl.ds(HN, HN)] = agm_buf[s].astype(jnp.float32)

        for s in range(P - 1):
            for sems, rsems, buf, nbr in (
                (agp_send, agp_recv, agp_buf, right),
                (agm_send, agm_recv, agm_buf, left),
            ):
                d = pltpu.make_async_remote_copy(
                    src_ref=buf.at[s],
                    dst_ref=buf.at[s],
                    send_sem=sems.at[s],
                    recv_sem=rsems.at[s],
                    device_id=(nbr,),
                    device_id_type=pl.DeviceIdType.MESH,
                )
                d.wait_send()

    nsem = P - 1
    return pl.pallas_call(
        body,
        out_shape=jax.ShapeDtypeStruct((m, n), jnp.float32),
        in_specs=[
            pl.BlockSpec(memory_space=pltpu.VMEM),
            pl.BlockSpec(memory_space=pltpu.VMEM),
        ],
        out_specs=pl.BlockSpec(memory_space=pltpu.VMEM),
        scratch_shapes=[
            pltpu.VMEM((M, k), F16),
            pltpu.VMEM((k, N), F16),
            pltpu.VMEM((P, CH, HN), F16),
            pltpu.VMEM((P, CH, HN), F16),
            pltpu.VMEM((nsem, CH, HN), F16),
            pltpu.VMEM((nsem, CH, HN), F16),
            pltpu.VMEM((nsem, CH, HN), F16),
            pltpu.VMEM((nsem, CH, HN), F16),
            pltpu.SemaphoreType.DMA((nsem,)),
            pltpu.SemaphoreType.DMA((nsem,)),
            pltpu.SemaphoreType.DMA((nsem,)),
            pltpu.SemaphoreType.DMA((nsem,)),
            pltpu.SemaphoreType.DMA((nsem,)),
            pltpu.SemaphoreType.DMA((nsem,)),
            pltpu.SemaphoreType.DMA((nsem,)),
            pltpu.SemaphoreType.DMA((nsem,)),
        ],
        compiler_params=pltpu.CompilerParams(
            collective_id=0,
            vmem_limit_bytes=100 * 1024 * 1024,
        ),
    )(A, B)


# device time: 243222 ns/iter; 2.0059x vs baseline; 1.5651x over previous
import jax
import jax.numpy as jnp
from jax import lax
from jax.experimental import pallas as pl
from jax.experimental.pallas import tpu as pltpu

P = 32
G = 8
Z = 4
M = 2048
N = 2048
HN = N // 2
RCH = M // G
SCH = RCH // Z
F16 = jnp.bfloat16
MESH = pl.DeviceIdType.MESH


def kernel(A, B):
    m, k = A.shape
    k2, n = B.shape

    def body(a_ref, b_ref, out_ref,
             ss1p, ss1m, r1p, r1m,
             ss2p, ss2m, r2p, r2m, a2p, a2m,
             s3p_own, s3m_own, r3p, r3m,
             s1p_s, s1p_r, s1m_s, s1m_r,
             s2p_s, s2p_r, s2m_s, s2m_r,
             g2p_s, g2p_r, g2m_s, g2m_r,
             s3p_s, s3p_r, s3m_s, s3m_r):
        my = lax.axis_index("i")
        q = lax.rem(my, G)
        g = lax.div(my, G)
        qR = g * G + lax.rem(q + 1, G)
        qL = g * G + lax.rem(q + G - 1, G)
        zR = lax.rem(g + 1, Z) * G + q
        zL = lax.rem(g + Z - 1, Z) * G + q

        def rows(j):
            return pl.ds(j * RCH, RCH)

        def subrows(j, u):
            return pl.ds(j * RCH + u * SCH, SCH)

        def mk(src, dst, ssem, rsem, dev):
            return pltpu.make_async_remote_copy(
                src_ref=src, dst_ref=dst, send_sem=ssem, recv_sem=rsem,
                device_id=(dev,), device_id_type=MESH,
            )

        out_ref[...] = jnp.dot(
            a_ref[...], b_ref[...], preferred_element_type=jnp.float32
        )
        ss1p[0] = out_ref[rows(q), pl.ds(0, HN)].astype(F16)
        ss1m[0] = out_ref[rows(q), pl.ds(HN, HN)].astype(F16)

        barrier = pltpu.get_barrier_semaphore()
        for nbr in (qL, qR, zL, zR):
            pl.semaphore_signal(barrier, inc=1, device_id=(nbr,),
                                device_id_type=MESH)
        pl.semaphore_wait(barrier, 4)

        for s in range(G - 1):
            rcp = lax.rem(q - s - 1 + G, G)
            rcm = lax.rem(q + s + 1, G)
            sp = mk(ss1p.at[s], r1p.at[s], s1p_s.at[s], s1p_r.at[s], qR)
            sm = mk(ss1m.at[s], r1m.at[s], s1m_s.at[s], s1m_r.at[s], qL)
            sp.start()
            sm.start()
            sp.wait_recv()
            accp = out_ref[rows(rcp), pl.ds(0, HN)] + r1p[s].astype(jnp.float32)
            ss1p[s + 1] = accp.astype(F16)
            if s == G - 2:
                out_ref[rows(rcp), pl.ds(0, HN)] = accp
            sm.wait_recv()
            accm = out_ref[rows(rcm), pl.ds(HN, HN)] + r1m[s].astype(jnp.float32)
            ss1m[s + 1] = accm.astype(F16)
            if s == G - 2:
                out_ref[rows(rcm), pl.ds(HN, HN)] = accm
        for s in range(G - 1):
            mk(ss1p.at[s], r1p.at[s], s1p_s.at[s], s1p_r.at[s], qR).wait_send()
            mk(ss1m.at[s], r1m.at[s], s1m_s.at[s], s1m_r.at[s], qL).wait_send()

        op = lax.rem(q + 1, G)
        om = lax.rem(q + G - 1, G)

        ss2p[0] = out_ref[subrows(op, g), pl.ds(0, HN)].astype(F16)
        ss2m[0] = out_ref[subrows(om, g), pl.ds(HN, HN)].astype(F16)
        for s in range(Z - 1):
            rup = lax.rem(g - s - 1 + Z, Z)
            rum = lax.rem(g + s + 1, Z)
            sp = mk(ss2p.at[s], r2p.at[s], s2p_s.at[s], s2p_r.at[s], zR)
            sm = mk(ss2m.at[s], r2m.at[s], s2m_s.at[s], s2m_r.at[s], zL)
            sp.start()
            sm.start()
            sp.wait_recv()
            accp = out_ref[subrows(op, rup), pl.ds(0, HN)] + r2p[s].astype(
                jnp.float32)
            ss2p[s + 1] = accp.astype(F16)
            if s == Z - 2:
                out_ref[subrows(op, rup), pl.ds(0, HN)] = accp
            sm.wait_recv()
            accm = out_ref[subrows(om, rum), pl.ds(HN, HN)] + r2m[s].astype(
                jnp.float32)
            ss2m[s + 1] = accm.astype(F16)
            if s == Z - 2:
                out_ref[subrows(om, rum), pl.ds(HN, HN)] = accm
        for s in range(Z - 1):
            mk(ss2p.at[s], r2p.at[s], s2p_s.at[s], s2p_r.at[s], zR).wait_send()
            mk(ss2m.at[s], r2m.at[s], s2m_s.at[s], s2m_r.at[s], zL).wait_send()

        for s in range(Z - 1):
            rup = lax.rem(g - s + Z, Z)
            rum = lax.rem(g + s, Z)
            srcp = ss2p.at[Z - 1] if s == 0 else a2p.at[s - 1]
            srcm = ss2m.at[Z - 1] if s == 0 else a2m.at[s - 1]
            sp = mk(srcp, a2p.at[s], g2p_s.at[s], g2p_r.at[s], zR)
            sm = mk(srcm, a2m.at[s], g2m_s.at[s], g2m_r.at[s], zL)
            sp.start()
            sm.start()
            sp.wait_recv()
            out_ref[subrows(op, rup), pl.ds(0, HN)] = a2p[s].astype(jnp.float32)
            sm.wait_recv()
            out_ref[subrows(om, rum), pl.ds(HN, HN)] = a2m[s].astype(jnp.float32)
        for s in range(Z - 1):
            mk(a2p.at[s], a2p.at[s], g2p_s.at[s], g2p_r.at[s], zR).wait_send()
            mk(a2m.at[s], a2m.at[s], g2m_s.at[s], g2m_r.at[s], zL).wait_send()

        s3p_own[...] = out_ref[rows(op), pl.ds(0, HN)].astype(F16)
        s3m_own[...] = out_ref[rows(om), pl.ds(HN, HN)].astype(F16)
        for s in range(G - 1):
            rcp = lax.rem(q - s + G, G)
            rcm = lax.rem(q + s, G)
            srcp = s3p_own if s == 0 else r3p.at[s - 1]
            srcm = s3m_own if s == 0 else r3m.at[s - 1]
            sp = mk(srcp, r3p.at[s], s3p_s.at[s], s3p_r.at[s], qR)
            sm = mk(srcm, r3m.at[s], s3m_s.at[s], s3m_r.at[s], qL)
            sp.start()
            sm.start()
            sp.wait_recv()
            out_ref[rows(rcp), pl.ds(0, HN)] = r3p[s].astype(jnp.float32)
            sm.wait_recv()
            out_ref[rows(rcm), pl.ds(HN, HN)] = r3m[s].astype(jnp.float32)
        for s in range(G - 1):
            mk(r3p.at[s], r3p.at[s], s3p_s.at[s], s3p_r.at[s], qR).wait_send()
            mk(r3m.at[s], r3m.at[s], s3m_s.at[s], s3m_r.at[s], qL).wait_send()

    dma = pltpu.SemaphoreType.DMA
    return pl.pallas_call(
        body,
        out_shape=jax.ShapeDtypeStruct((m, n), jnp.float32),
        in_specs=[
            pl.BlockSpec(memory_space=pltpu.VMEM),
            pl.BlockSpec(memory_space=pltpu.VMEM),
        ],
        out_specs=pl.BlockSpec(memory_space=pltpu.VMEM),
        scratch_shapes=[
            pltpu.VMEM((G, RCH, HN), F16),
            pltpu.VMEM((G, RCH, HN), F16),
            pltpu.VMEM((G - 1, RCH, HN), F16),
            pltpu.VMEM((G - 1, RCH, HN), F16),
            pltpu.VMEM((Z, SCH, HN), F16),
            pltpu.VMEM((Z, SCH, HN), F16),
            pltpu.VMEM((Z - 1, SCH, HN), F16),
            pltpu.VMEM((Z - 1, SCH, HN), F16),
            pltpu.VMEM((Z - 1, SCH, HN), F16),
            pltpu.VMEM((Z - 1, SCH, HN), F16),
            pltpu.VMEM((RCH, HN), F16),
            pltpu.VMEM((RCH, HN), F16),
            pltpu.VMEM((G - 1, RCH, HN), F16),
            pltpu.VMEM((G - 1, RCH, HN), F16),
            dma((G - 1,)), dma((G - 1,)), dma((G - 1,)), dma((G - 1,)),
            dma((Z - 1,)), dma((Z - 1,)), dma((Z - 1,)), dma((Z - 1,)),
            dma((Z - 1,)), dma((Z - 1,)), dma((Z - 1,)), dma((Z - 1,)),
            dma((G - 1,)), dma((G - 1,)), dma((G - 1,)), dma((G - 1,)),
        ],
        compiler_params=pltpu.CompilerParams(
            collective_id=0,
            vmem_limit_bytes=100 * 1024 * 1024,
        ),
    )(A, B)
